# baseline (speedup 1.0000x reference)
import functools

import jax
import jax.numpy as jnp
from jax import lax
from jax.experimental import pallas as pl
from jax.experimental.pallas import tpu as pltpu
from jax.experimental.pallas import tpu_sc as plsc

N = 10000
V = 512
E = 160000
B = 16
H = 128
HEADS = 4
DH = H // HEADS
TIME_DIM = 128

NW = 32
E_PAD = 163840
N_PAD = 10240
W_NARROW = 16
W_COMB = H + 8
ET = 1024
NT = 1000


def _silu(x):
    return x / (1.0 + jnp.exp(-x))


def _mlp2k(x, w1, b1, w2, b2):
    return jnp.dot(_silu(jnp.dot(x, w1) + b1), w2) + b2


def _ln_k(x, g, b):
    m = jnp.mean(x, axis=-1, keepdims=True)
    v = jnp.mean((x - m) * (x - m), axis=-1, keepdims=True)
    return (x - m) / jnp.sqrt(v + 1e-5) * g + b


def _full(shape):
    nd = len(shape)
    return pl.BlockSpec(shape, lambda *a: (0,) * nd)


def _rows(T, shape):
    nd = len(shape)
    return pl.BlockSpec((T,) + tuple(shape[1:]),
                        lambda i: (i,) + (0,) * (nd - 1))


@functools.lru_cache(maxsize=None)
def _gather_kernel(n_pad, tab_rows, w, c, nb):
    per_w = n_pad // NW
    nchunks = per_w // c
    mesh = plsc.VectorSubcoreMesh(core_axis_name="c", subcore_axis_name="s")

    @functools.partial(
        pl.kernel, mesh=mesh,
        out_type=jax.ShapeDtypeStruct((n_pad, w), jnp.float32),
        scratch_types=[
            pltpu.VMEM((nchunks, c), jnp.int32),
            pltpu.VMEM((nb, c, w), jnp.float32),
        ] + [pltpu.SemaphoreType.DMA] * (nb + 1),
        name=f"sc_gather_{tab_rows}x{w}",
    )
    def k(table_hbm, idx_hbm, out_hbm, idx_v, rows_v, *sems):
        sg = sems[:nb]
        sw = sems[nb]
        wid = lax.axis_index("s") * 2 + lax.axis_index("c")
        pltpu.sync_copy(idx_hbm.at[pl.ds(wid * nchunks, nchunks)], idx_v)
        base = wid * per_w
        for b in range(nb):
            pltpu.async_copy(table_hbm.at[idx_v.at[b]], rows_v.at[b], sg[b])

        def outer(i, carry):
            j0 = i * nb
            for b in range(nb):
                j = j0 + b
                pltpu.make_async_copy(table_hbm.at[pl.ds(0, c)],
                                      rows_v.at[b], sg[b]).wait()
                pltpu.async_copy(rows_v.at[b],
                                 out_hbm.at[pl.ds(base + j * c, c)],
                                 sw).wait()

                @pl.when(j + nb < nchunks)
                def _():
                    pltpu.async_copy(table_hbm.at[idx_v.at[j + nb]],
                                     rows_v.at[b], sg[b])
            return carry

        lax.fori_loop(0, nchunks // nb, outer, 0)

    return k


def _gather(table, idx2d):
    table = jnp.asarray(table, jnp.float32)
    w = table.shape[1]
    nb = 1 if w > 256 else (2 if w > 128 else 4)
    k = _gather_kernel(idx2d.size, table.shape[0], w, idx2d.shape[1], nb)
    return k(table, idx2d)


def _rpad(r):
    return -(-r // 128) * 128


@functools.lru_cache(maxsize=None)
def _scatter_kernel(n_pad, r_pad, w, c):
    per_w = n_pad // NW
    nchunks = per_w // c
    rps = r_pad // 16
    mesh = plsc.VectorSubcoreMesh(core_axis_name="c", subcore_axis_name="s")

    @functools.partial(
        pl.kernel, mesh=mesh,
        out_type=jax.ShapeDtypeStruct((2, r_pad, w), jnp.float32),
        scratch_types=[
            pltpu.VMEM((c,), jnp.int32),
            pltpu.VMEM((c, w), jnp.float32),
            pltpu.VMEM_SHARED((r_pad, w), jnp.float32),
        ],
        name=f"sc_scatter_{r_pad}x{w}",
    )
    def k(rows_hbm, idx_hbm, zeros_hbm, out_hbm, idx_v, rows_v, shared):
        cid = lax.axis_index("c")
        sid = lax.axis_index("s")
        wid = sid * 2 + cid
        base = wid * per_w
        pltpu.sync_copy(zeros_hbm.at[pl.ds(sid * rps, rps)],
                        shared.at[pl.ds(sid * rps, rps)])
        plsc.subcore_barrier()

        def step(j, carry):
            pltpu.sync_copy(idx_hbm.at[wid * nchunks + j], idx_v)
            pltpu.sync_copy(rows_hbm.at[pl.ds(base + j * c, c)], rows_v)
            pltpu.sync_copy(rows_v, shared.at[idx_v], add=True)
            return carry

        lax.fori_loop(0, nchunks, step, 0)
        plsc.subcore_barrier()
        pltpu.sync_copy(shared.at[pl.ds(sid * rps, rps)],
                        out_hbm.at[cid, pl.ds(sid * rps, rps)])

    return k


def _scatter_add(rows, idx2d, r):
    w = rows.shape[1]
    rp = _rpad(r)
    k = _scatter_kernel(idx2d.size, rp, w, idx2d.shape[1])
    return k(rows, idx2d, jnp.zeros((rp, w), jnp.float32))


@functools.lru_cache(maxsize=None)
def _scatter_small_kernel(n_pad, r_pad, w, c):
    nchunks = n_pad // NW // c
    per_w = n_pad // NW
    rps = r_pad // 16
    mesh = plsc.VectorSubcoreMesh(core_axis_name="c", subcore_axis_name="s")

    @functools.partial(
        pl.kernel, mesh=mesh,
        out_type=jax.ShapeDtypeStruct((2, r_pad, w), jnp.float32),
        scratch_types=[
            pltpu.VMEM((c,), jnp.int32),
            pltpu.VMEM((c, w), jnp.float32),
            pltpu.VMEM_SHARED((r_pad, w), jnp.float32),
        ],
        name=f"sc_scatter_s_{r_pad}x{w}",
    )
    def k(rows_hbm, idx_hbm, zeros_hbm, out_hbm, idx_v, rows_v, shared):
        cid = lax.axis_index("c")
        sid = lax.axis_index("s")
        wid = sid * 2 + cid
        base = wid * per_w
        pltpu.sync_copy(zeros_hbm.at[pl.ds(sid * rps, rps)],
                        shared.at[pl.ds(sid * rps, rps)])
        plsc.subcore_barrier()

        def step(j, carry):
            off = base + j * c
            pltpu.sync_copy(idx_hbm.at[pl.ds(off, c)], idx_v)
            pltpu.sync_copy(rows_hbm.at[pl.ds(off, c)], rows_v)
            pltpu.sync_copy(rows_v, shared.at[idx_v], add=True)
            return carry

        lax.fori_loop(0, nchunks, step, 0)
        plsc.subcore_barrier()
        pltpu.sync_copy(shared.at[pl.ds(sid * rps, rps)],
                        out_hbm.at[cid, pl.ds(sid * rps, rps)])

    return k


def _scatter_add_small(rows, idx_pad, r, c=64):
    w = rows.shape[1]
    rp = _rpad(r)
    k = _scatter_small_kernel(idx_pad.shape[0], rp, w, c)
    return k(rows, idx_pad, jnp.zeros((rp, w), jnp.float32))


def _k_smalls(t_ref, fr_ref, gf_ref, vf_ref,
              tw1, tb1, tw2, tb2, gw1, gb1, gw2, gb2, vw1, vb1, vw2, vb2,
              temb_o, hg_o, hv_o):
    ang = t_ref[...] * fr_ref[...]
    traw = jnp.concatenate([jnp.sin(ang), jnp.cos(ang)], axis=-1)
    temb_o[...] = _mlp2k(traw, tw1[...], tb1[...], tw2[...], tb2[...])
    hg_o[...] = _mlp2k(gf_ref[...], gw1[...], gb1[...], gw2[...], gb2[...])
    hv_o[...] = _mlp2k(vf_ref[...], vw1[...], vb1[...], vw2[...], vb2[...])


def _k_mlp2(x_ref, w1, b1, w2, b2, o_ref):
    o_ref[...] = _mlp2k(x_ref[...], w1[...], b1[...], w2[...], b2[...])


def _k_eproj(ea_ref, xt_ref, w1a, w1b, b1, w2, b2, o_ref):
    x = jnp.dot(ea_ref[...], w1a[...]) + xt_ref[...] * w1b[...] + b1[...]
    o_ref[...] = jnp.dot(_silu(x), w2[...]) + b2[...]


def _k_edyn(ed_ref, pw1, pb1, pw2, pb2, bw1, bb1, bw2, bb2, ed_o, bias_o):
    x = ed_ref[...]
    ed_o[...] = _mlp2k(x, pw1[...], pb1[...], pw2[...], pb2[...])
    bias_o[...] = _mlp2k(x, bw1[...], bb1[...], bw2[...], bb2[...])


def _k_mm(x_ref, w_ref, o_ref):
    o_ref[...] = jnp.dot(x_ref[...], w_ref[...])


def _onehot(ints, n):
    return (ints == lax.broadcasted_iota(jnp.int32, (1, n), 1)
            ).astype(jnp.float32)


def _attn_edge_core(qg, kvg, he, wk, wv, e_o, wv_o):
    T = qg.shape[0]
    ke = kvg[:, :H] + jnp.dot(he, wk)
    prod = qg * ke
    s = jnp.sum(prod.reshape(T, HEADS, DH), axis=-1) / jnp.sqrt(float(DH))
    row = pl.program_id(0) * T + lax.broadcasted_iota(jnp.int32, (T, 1), 0)
    ex = jnp.where(row < E, jnp.exp(s), 0.0)
    ve = kvg[:, H:2 * H] + jnp.dot(he, wv)
    wv_o[...] = (ve.reshape(T, HEADS, DH) * ex[:, :, None]).reshape(T, H)
    e_o[...] = jnp.concatenate(
        [ex, jnp.zeros((T, W_NARROW - HEADS), jnp.float32)], axis=-1)


def _k_attn_edge_sv(qg_ref, kvtab_ref, si_ref, he_ref, wk_ref, wv_ref,
                    e_o, wv_o):
    oh = _onehot(si_ref[...], kvtab_ref.shape[0])
    kvg = jnp.dot(oh, kvtab_ref[...])
    _attn_edge_core(qg_ref[...], kvg, he_ref[...],
                    wk_ref[...], wv_ref[...], e_o, wv_o)


def _k_attn_edge_sq(qtab_ref, si_ref, kvg_ref, he_ref, wk_ref, wv_ref,
                    e_o, wv_o):
    oh = _onehot(si_ref[...], qtab_ref.shape[0])
    qg = jnp.dot(oh, qtab_ref[...])
    _attn_edge_core(qg, kvg_ref[...], he_ref[...],
                    wk_ref[...], wv_ref[...], e_o, wv_o)


def _k_comb(aggp_ref, zp_ref, hprev_ref, wo_ref, g_ref, b_ref, o_ref):
    T = hprev_ref.shape[0]
    aggp = aggp_ref[...]
    zp = zp_ref[...]
    z = (zp[0] + zp[1])[:, :HEADS]
    s = aggp[0] + aggp[1]
    agg = (s.reshape(T, HEADS, DH) / (z[:, :, None] + 1e-9)).reshape(T, H)
    h = hprev_ref[...] + jnp.dot(agg, wo_ref[...])
    o_ref[...] = _ln_k(h, g_ref[...], b_ref[...])


def _k_delta(hvtab_ref, si_ref, hnd_ref, he_ref, ed_ref, bias_ref,
             w1a, w1b, w1c, w1d, w1e, b1, w2, b2, o_ref):
    oh = _onehot(si_ref[...], hvtab_ref.shape[0])
    tv = jnp.dot(hvtab_ref[...], w1a[...])
    x = (jnp.dot(oh, tv) + jnp.dot(hnd_ref[...], w1b[...])
         + jnp.dot(he_ref[...], w1c[...]) + jnp.dot(ed_ref[...], w1d[...])
         + bias_ref[...] * w1e[...] + b1[...])
    o_ref[...] = bias_ref[...] + jnp.dot(_silu(x), w2[...]) + b2[...]


def _k_glob(hg_ref, pooledp_ref, cntp_ref, temb_ref,
            w1a, w1b, w1c, b1, w2, b2, o_ref):
    pooledp = pooledp_ref[...]
    cntp = cntp_ref[...]
    cnt = (cntp[0] + cntp[1])[:, :1] + 1e-6
    pooled = (pooledp[0] + pooledp[1]) / cnt
    hg = hg_ref[...]
    x = (jnp.dot(hg, w1a[...]) + jnp.dot(pooled, w1b[...])
         + jnp.dot(temb_ref[...], w1c[...]) + b1[...])
    o_ref[...] = hg + jnp.dot(_silu(x), w2[...]) + b2[...]


def _k_final(hnd_ref, hvtab_ref, si_ref, he_ref, edp_ref, gt_ref, gi_ref,
             bias_ref, lng, lnb, w1, b1, w2, b2, w3, b3, o_ref):
    he = _ln_k(he_ref[...], lng[...], lnb[...])
    hnd = hnd_ref[...]
    hvs = jnp.dot(_onehot(si_ref[...], hvtab_ref.shape[0]), hvtab_ref[...])
    w = w1[...]
    g2 = jnp.dot(gt_ref[...], w[512:768])
    x = (jnp.dot(hnd, w[0:128]) + jnp.dot(hvs, w[128:256])
         + jnp.dot(he, w[256:384]) + jnp.dot(edp_ref[...], w[384:512])
         + jnp.dot(_onehot(gi_ref[...], B), g2)
         + jnp.dot(hnd * hvs, w[768:896]) + b1[...])
    x = _silu(x)
    x = _silu(jnp.dot(x, w2[...]) + b2[...])
    o_ref[...] = jnp.dot(x, w3[...]) + b3[...] + bias_ref[...]


def _r1(x):
    return jnp.asarray(x, jnp.float32).reshape(1, -1)


def _mlp2_args(p):
    return (p["w1"], _r1(p["b1"]), p["w2"], _r1(p["b2"]))


def kernel(node_feat, veh_feat, edge_attr, xt01, t, graph_feat, edge_dyn,
           params, src, dst, edge_graph, node_batch):
    p = params
    f32 = jnp.float32
    src = src.astype(jnp.int32)
    dst = dst.astype(jnp.int32)
    edge_graph = edge_graph.astype(jnp.int32)
    node_batch = node_batch.astype(jnp.int32)

    def padE(a):
        pw = ((0, E_PAD - E),) + ((0, 0),) * (a.ndim - 1)
        return jnp.pad(a, pw)

    srcp = padE(src).reshape(-1, 128)
    dstp = padE(dst).reshape(-1, 128)
    srci = padE(src).reshape(E_PAD, 1)
    egi = padE(edge_graph).reshape(E_PAD, 1)
    eap = padE(jnp.asarray(edge_attr, f32))
    xtp = padE(jnp.asarray(xt01, f32)).reshape(E_PAD, 1)
    edp = padE(jnp.asarray(edge_dyn, f32))
    nbp = jnp.pad(node_batch, (0, N_PAD - N))
    ones_rows = jnp.zeros((N_PAD, W_NARROW), f32).at[:N, :].set(1.0)

    tcol = jnp.asarray(t, f32).reshape(B, 1)
    half = TIME_DIM // 2
    freqs = jnp.exp(-jnp.log(10000.0)
                    * jnp.arange(half, dtype=f32) / float(half)).reshape(1, half)

    temb, h_g, h_v = pl.pallas_call(
        _k_smalls,
        in_specs=[_full((B, 1)), _full((1, half)), _full((B, 8)),
                  _full((V, 8)),
                  _full((TIME_DIM, H)), _full((1, H)), _full((H, H)), _full((1, H)),
                  _full((8, H)), _full((1, H)), _full((H, H)), _full((1, H)),
                  _full((8, H)), _full((1, H)), _full((H, H)), _full((1, H))],
        out_specs=[_full((B, H)), _full((B, H)), _full((V, H))],
        out_shape=[jax.ShapeDtypeStruct((B, H), f32),
                   jax.ShapeDtypeStruct((B, H), f32),
                   jax.ShapeDtypeStruct((V, H), f32)],
    )(tcol, freqs, jnp.asarray(graph_feat, f32), jnp.asarray(veh_feat, f32),
      *_mlp2_args(p["time_proj"]), *_mlp2_args(p["global_proj"]),
      *_mlp2_args(p["veh_proj"]))

    h_n = pl.pallas_call(
        _k_mlp2,
        grid=(N // NT,),
        in_specs=[_rows(NT, (N, 8)), _full((8, H)), _full((1, H)),
                  _full((H, H)), _full((1, H))],
        out_specs=_rows(NT, (N, H)),
        out_shape=jax.ShapeDtypeStruct((N, H), f32),
    )(jnp.asarray(node_feat, f32), *_mlp2_args(p["node_proj"]))

    ep = p["edge_proj"]
    h_e = pl.pallas_call(
        _k_eproj,
        grid=(E_PAD // ET,),
        in_specs=[_rows(ET, (E_PAD, 4)), _rows(ET, (E_PAD, 1)),
                  _full((4, H)), _full((1, H)), _full((1, H)),
                  _full((H, H)), _full((1, H))],
        out_specs=_rows(ET, (E_PAD, H)),
        out_shape=jax.ShapeDtypeStruct((E_PAD, H), f32),
    )(eap, xtp, ep["w1"][:4], _r1(ep["w1"][4]), _r1(ep["b1"]),
      ep["w2"], _r1(ep["b2"]))

    bp = p["edge_bias_mlp"]
    e_dyn, bias = pl.pallas_call(
        _k_edyn,
        grid=(E_PAD // ET,),
        in_specs=[_rows(ET, (E_PAD, 7)),
                  _full((7, H)), _full((1, H)), _full((H, H)), _full((1, H)),
                  _full((7, H // 2)), _full((1, H // 2)),
                  _full((H // 2, 1)), _full((1, 1))],
        out_specs=[_rows(ET, (E_PAD, H)), _rows(ET, (E_PAD, 1))],
        out_shape=[jax.ShapeDtypeStruct((E_PAD, H), f32),
                   jax.ShapeDtypeStruct((E_PAD, 1), f32)],
    )(edp, *_mlp2_args(p["edge_dyn_proj"]), *_mlp2_args(bp))

    cntp = _scatter_add_small(ones_rows, nbp, B, c=64)

    def _mm_t(x, w):
        r0, wb = x.shape[0], w.shape[1]
        rt0 = min(r0, NT)
        return pl.pallas_call(
            _k_mm, grid=(r0 // rt0,),
            in_specs=[_rows(rt0, (r0, H)), _full((H, wb))],
            out_specs=_rows(rt0, (r0, wb)),
            out_shape=jax.ShapeDtypeStruct((r0, wb), f32),
        )(x, w)

    def attn(h_dst_tab, small_tab, wide_arr, wide_spec, q_idx, r,
             ap, lnp, si, sv):
        rt = min(r, NT)
        outs = [_rows(ET, (E_PAD, W_NARROW)), _rows(ET, (E_PAD, H))]
        oshape = [jax.ShapeDtypeStruct((E_PAD, W_NARROW), f32),
                  jax.ShapeDtypeStruct((E_PAD, H), f32)]
        if sv:
            e16, wvrows = pl.pallas_call(
                _k_attn_edge_sv, grid=(E_PAD // ET,),
                in_specs=[wide_spec, _full((V, 2 * H)),
                          _rows(ET, (E_PAD, 1)), _rows(ET, (E_PAD, H)),
                          _full((H, H)), _full((H, H))],
                out_specs=outs, out_shape=oshape,
            )(wide_arr, small_tab, si, h_e, ap["wk"], ap["wv"])
        else:
            e16, wvrows = pl.pallas_call(
                _k_attn_edge_sq, grid=(E_PAD // ET,),
                in_specs=[_full((V, H)), _rows(ET, (E_PAD, 1)),
                          wide_spec, _rows(ET, (E_PAD, H)),
                          _full((H, H)), _full((H, H))],
                out_specs=outs, out_shape=oshape,
            )(small_tab, si, wide_arr, h_e, ap["wk"], ap["wv"])

        zp = _scatter_add(e16, q_idx, r)
        aggp = _scatter_add(wvrows, q_idx, r)

        h_new = pl.pallas_call(
            _k_comb, grid=(r // rt,),
            in_specs=[pl.BlockSpec((2, rt, H), lambda i: (0, i, 0)),
                      pl.BlockSpec((2, rt, W_NARROW), lambda i: (0, i, 0)),
                      _rows(rt, (r, H)), _full((H, H)),
                      _full((1, H)), _full((1, H))],
            out_specs=_rows(rt, (r, H)),
            out_shape=jax.ShapeDtypeStruct((r, H), f32),
        )(aggp, zp, h_dst_tab, ap["wo"], _r1(lnp["g"]), _r1(lnp["b"]))
        return h_new

    layers = p["layers"]
    nl = len(layers)
    eyeH = jnp.eye(H, dtype=f32)
    qg_a = _gather(_mm_t(h_n, layers[0]["v2n"]["wq"]), dstp)
    qg_s = _rows(ET, (E_PAD, H))
    for li, lp in enumerate(layers):
        ap1, ap2 = lp["v2n"], lp["n2v"]
        kvtab_v = _mm_t(h_v, jnp.concatenate([ap1["wk"], ap1["wv"]], axis=1))
        h_n = attn(h_n, kvtab_v, qg_a, qg_s, dstp, N,
                   ap1, lp["ln_n"], srci, True)

        kvgg = _gather(
            _mm_t(h_n, jnp.concatenate([ap2["wk"], ap2["wv"]], axis=1)),
            dstp)
        if li + 1 < nl:
            aux = _mm_t(h_n, jnp.concatenate(
                [eyeH, layers[li + 1]["v2n"]["wq"]], axis=1))
        else:
            aux = h_n
        auxg = _gather(aux, dstp)
        hnd_s = pl.BlockSpec((ET, H), lambda i: (i, 0))

        qtab_v = _mm_t(h_v, ap2["wq"])
        h_v = attn(h_v, qtab_v, kvgg, _rows(ET, (E_PAD, 2 * H)),
                   srcp, V, ap2, lp["ln_v"], srci, False)
        if li + 1 < nl:
            qg_a = auxg
            qg_s = pl.BlockSpec((ET, H), lambda i: (i, 1))

        dp = lp["edge_delta"]
        w1 = dp["w1"]
        bias = pl.pallas_call(
            _k_delta, grid=(E_PAD // ET,),
            in_specs=[_full((V, H)), _rows(ET, (E_PAD, 1)),
                      hnd_s,
                      _rows(ET, (E_PAD, H)), _rows(ET, (E_PAD, 7)),
                      _rows(ET, (E_PAD, 1)),
                      _full((H, H // 2)), _full((H, H // 2)),
                      _full((H, H // 2)), _full((7, H // 2)),
                      _full((1, H // 2)), _full((1, H // 2)),
                      _full((H // 2, 1)), _full((1, 1))],
            out_specs=_rows(ET, (E_PAD, 1)),
            out_shape=jax.ShapeDtypeStruct((E_PAD, 1), f32),
        )(h_v, srci, auxg, h_e, edp, bias,
          w1[0:H], w1[H:2 * H], w1[2 * H:3 * H], w1[3 * H:3 * H + 7],
          _r1(w1[3 * H + 7]), _r1(dp["b1"]), dp["w2"], _r1(dp["b2"]))

        hn_pad = jnp.pad(h_n, ((0, N_PAD - N), (0, 0)))
        pooledp = _scatter_add_small(hn_pad, nbp, B, c=64)
        gp = lp["glob"]
        gw1 = gp["w1"]
        h_g = pl.pallas_call(
            _k_glob,
            grid=(1,),
            in_specs=[_full((B, H)), _full((2, B, H)),
                      _full((2, B, W_NARROW)), _full((B, H)),
                      _full((H, H)), _full((H, H)), _full((H, H)),
                      _full((1, H)), _full((H, H)), _full((1, H))],
            out_specs=_full((B, H)),
            out_shape=jax.ShapeDtypeStruct((B, H), f32),
        )(h_g, pooledp, cntp, temb,
          gw1[0:H], gw1[H:2 * H], gw1[2 * H:3 * H], _r1(gp["b1"]),
          gp["w2"], _r1(gp["b2"]))

    gt_tab = jnp.concatenate([h_g, temb], axis=1)
    ph = p["edge_head"]
    oln = p["edge_out_ln"]
    logits = pl.pallas_call(
        _k_final, grid=(E_PAD // ET,),
        in_specs=[hnd_s, _full((V, H)),
                  _rows(ET, (E_PAD, 1)),
                  _rows(ET, (E_PAD, H)), _rows(ET, (E_PAD, H)),
                  _full((B, 2 * H)), _rows(ET, (E_PAD, 1)),
                  _rows(ET, (E_PAD, 1)),
                  _full((1, H)), _full((1, H)),
                  _full((7 * H, H)), _full((1, H)),
                  _full((H, H // 2)), _full((1, H // 2)),
                  _full((H // 2, 1)), _full((1, 1))],
        out_specs=_rows(ET, (E_PAD, 1)),
        out_shape=jax.ShapeDtypeStruct((E_PAD, 1), f32),
    )(auxg, h_v, srci, h_e, e_dyn, gt_tab, egi, bias,
      _r1(oln["g"]), _r1(oln["b"]), ph["w1"], _r1(ph["b1"]),
      ph["w2"], _r1(ph["b2"]), ph["w3"], _r1(ph["b3"]))

    return logits[:E, 0]


# speedup vs baseline: 1.0510x; 1.0510x over previous
import functools

import jax
import jax.numpy as jnp
from jax import lax
from jax.experimental import pallas as pl
from jax.experimental.pallas import tpu as pltpu
from jax.experimental.pallas import tpu_sc as plsc

N = 10000
V = 512
E = 160000
B = 16
H = 128
HEADS = 4
DH = H // HEADS
TIME_DIM = 128

NW = 32
E_PAD = 163840
N_PAD = 10240
W_NARROW = 16
W_COMB = H + 8
ET = 1024
NT = 1000


def _silu(x):
    return x / (1.0 + jnp.exp(-x))


def _mlp2k(x, w1, b1, w2, b2):
    return jnp.dot(_silu(jnp.dot(x, w1) + b1), w2) + b2


def _ln_k(x, g, b):
    m = jnp.mean(x, axis=-1, keepdims=True)
    v = jnp.mean((x - m) * (x - m), axis=-1, keepdims=True)
    return (x - m) / jnp.sqrt(v + 1e-5) * g + b


def _full(shape):
    nd = len(shape)
    return pl.BlockSpec(shape, lambda *a: (0,) * nd)


def _rows(T, shape):
    nd = len(shape)
    return pl.BlockSpec((T,) + tuple(shape[1:]),
                        lambda i: (i,) + (0,) * (nd - 1))


@functools.lru_cache(maxsize=None)
def _gather_kernel(n_pad, tab_rows, w, c, nb):
    per_w = n_pad // NW
    nchunks = per_w // c
    mesh = plsc.VectorSubcoreMesh(core_axis_name="c", subcore_axis_name="s")

    @functools.partial(
        pl.kernel, mesh=mesh,
        out_type=jax.ShapeDtypeStruct((n_pad, w), jnp.float32),
        scratch_types=[
            pltpu.VMEM((nchunks, c), jnp.int32),
            pltpu.VMEM((nb, c, w), jnp.float32),
        ] + [pltpu.SemaphoreType.DMA] * (nb + 1),
        name=f"sc_gather_{tab_rows}x{w}",
    )
    def k(table_hbm, idx_hbm, out_hbm, idx_v, rows_v, *sems):
        sg = sems[:nb]
        sw = sems[nb]
        wid = lax.axis_index("s") * 2 + lax.axis_index("c")
        pltpu.sync_copy(idx_hbm.at[pl.ds(wid * nchunks, nchunks)], idx_v)
        base = wid * per_w
        for b in range(nb):
            pltpu.async_copy(table_hbm.at[idx_v.at[b]], rows_v.at[b], sg[b])

        def outer(i, carry):
            j0 = i * nb
            for b in range(nb):
                j = j0 + b
                pltpu.make_async_copy(table_hbm.at[pl.ds(0, c)],
                                      rows_v.at[b], sg[b]).wait()
                pltpu.async_copy(rows_v.at[b],
                                 out_hbm.at[pl.ds(base + j * c, c)],
                                 sw).wait()

                @pl.when(j + nb < nchunks)
                def _():
                    pltpu.async_copy(table_hbm.at[idx_v.at[j + nb]],
                                     rows_v.at[b], sg[b])
            return carry

        lax.fori_loop(0, nchunks // nb, outer, 0)

    return k


def _gather(table, idx2d):
    table = jnp.asarray(table, jnp.float32)
    w = table.shape[1]
    c = idx2d.shape[1]
    nb = max(1, min(4, 262144 // (c * w * 4)))
    k = _gather_kernel(idx2d.size, table.shape[0], w, idx2d.shape[1], nb)
    return k(table, idx2d)


def _rpad(r):
    return -(-r // 128) * 128


@functools.lru_cache(maxsize=None)
def _scatter_kernel(n_pad, r_pad, w, c):
    per_w = n_pad // NW
    nchunks = per_w // c
    rps = r_pad // 16
    mesh = plsc.VectorSubcoreMesh(core_axis_name="c", subcore_axis_name="s")

    @functools.partial(
        pl.kernel, mesh=mesh,
        out_type=jax.ShapeDtypeStruct((2, r_pad, w), jnp.float32),
        scratch_types=[
            pltpu.VMEM((c,), jnp.int32),
            pltpu.VMEM((c, w), jnp.float32),
            pltpu.VMEM_SHARED((r_pad, w), jnp.float32),
        ],
        name=f"sc_scatter_{r_pad}x{w}",
    )
    def k(rows_hbm, idx_hbm, zeros_hbm, out_hbm, idx_v, rows_v, shared):
        cid = lax.axis_index("c")
        sid = lax.axis_index("s")
        wid = sid * 2 + cid
        base = wid * per_w
        pltpu.sync_copy(zeros_hbm.at[pl.ds(sid * rps, rps)],
                        shared.at[pl.ds(sid * rps, rps)])
        plsc.subcore_barrier()

        def step(j, carry):
            pltpu.sync_copy(idx_hbm.at[wid * nchunks + j], idx_v)
            pltpu.sync_copy(rows_hbm.at[pl.ds(base + j * c, c)], rows_v)
            pltpu.sync_copy(rows_v, shared.at[idx_v], add=True)
            return carry

        lax.fori_loop(0, nchunks, step, 0)
        plsc.subcore_barrier()
        pltpu.sync_copy(shared.at[pl.ds(sid * rps, rps)],
                        out_hbm.at[cid, pl.ds(sid * rps, rps)])

    return k


def _scatter_add(rows, idx2d, r):
    w = rows.shape[1]
    rp = _rpad(r)
    k = _scatter_kernel(idx2d.size, rp, w, idx2d.shape[1])
    return k(rows, idx2d, jnp.zeros((rp, w), jnp.float32))


@functools.lru_cache(maxsize=None)
def _scatter_small_kernel(n_pad, r_pad, w, c):
    nchunks = n_pad // NW // c
    per_w = n_pad // NW
    rps = r_pad // 16
    mesh = plsc.VectorSubcoreMesh(core_axis_name="c", subcore_axis_name="s")

    @functools.partial(
        pl.kernel, mesh=mesh,
        out_type=jax.ShapeDtypeStruct((2, r_pad, w), jnp.float32),
        scratch_types=[
            pltpu.VMEM((c,), jnp.int32),
            pltpu.VMEM((c, w), jnp.float32),
            pltpu.VMEM_SHARED((r_pad, w), jnp.float32),
        ],
        name=f"sc_scatter_s_{r_pad}x{w}",
    )
    def k(rows_hbm, idx_hbm, zeros_hbm, out_hbm, idx_v, rows_v, shared):
        cid = lax.axis_index("c")
        sid = lax.axis_index("s")
        wid = sid * 2 + cid
        base = wid * per_w
        pltpu.sync_copy(zeros_hbm.at[pl.ds(sid * rps, rps)],
                        shared.at[pl.ds(sid * rps, rps)])
        plsc.subcore_barrier()

        def step(j, carry):
            off = base + j * c
            pltpu.sync_copy(idx_hbm.at[pl.ds(off, c)], idx_v)
            pltpu.sync_copy(rows_hbm.at[pl.ds(off, c)], rows_v)
            pltpu.sync_copy(rows_v, shared.at[idx_v], add=True)
            return carry

        lax.fori_loop(0, nchunks, step, 0)
        plsc.subcore_barrier()
        pltpu.sync_copy(shared.at[pl.ds(sid * rps, rps)],
                        out_hbm.at[cid, pl.ds(sid * rps, rps)])

    return k


def _scatter_add_small(rows, idx_pad, r, c=64):
    w = rows.shape[1]
    rp = _rpad(r)
    k = _scatter_small_kernel(idx_pad.shape[0], rp, w, c)
    return k(rows, idx_pad, jnp.zeros((rp, w), jnp.float32))


def _k_smalls(t_ref, fr_ref, gf_ref, vf_ref,
              tw1, tb1, tw2, tb2, gw1, gb1, gw2, gb2, vw1, vb1, vw2, vb2,
              temb_o, hg_o, hv_o):
    ang = t_ref[...] * fr_ref[...]
    traw = jnp.concatenate([jnp.sin(ang), jnp.cos(ang)], axis=-1)
    temb_o[...] = _mlp2k(traw, tw1[...], tb1[...], tw2[...], tb2[...])
    hg_o[...] = _mlp2k(gf_ref[...], gw1[...], gb1[...], gw2[...], gb2[...])
    hv_o[...] = _mlp2k(vf_ref[...], vw1[...], vb1[...], vw2[...], vb2[...])


def _k_mlp2(x_ref, w1, b1, w2, b2, o_ref):
    o_ref[...] = _mlp2k(x_ref[...], w1[...], b1[...], w2[...], b2[...])


def _k_eproj(ea_ref, xt_ref, w1a, w1b, b1, w2, b2, o_ref):
    x = jnp.dot(ea_ref[...], w1a[...]) + xt_ref[...] * w1b[...] + b1[...]
    o_ref[...] = jnp.dot(_silu(x), w2[...]) + b2[...]


def _k_edyn(ed_ref, pw1, pb1, pw2, pb2, bw1, bb1, bw2, bb2, ed_o, bias_o):
    x = ed_ref[...]
    ed_o[...] = _mlp2k(x, pw1[...], pb1[...], pw2[...], pb2[...])
    bias_o[...] = _mlp2k(x, bw1[...], bb1[...], bw2[...], bb2[...])


def _k_mm(x_ref, w_ref, o_ref):
    o_ref[...] = jnp.dot(x_ref[...], w_ref[...])


def _onehot(ints, n):
    return (ints == lax.broadcasted_iota(jnp.int32, (1, n), 1)
            ).astype(jnp.float32)


def _attn_edge_core(qg, kvg, he, wk, wv, e_o, wv_o):
    T = qg.shape[0]
    ke = kvg[:, :H] + jnp.dot(he, wk)
    prod = qg * ke
    s = jnp.sum(prod.reshape(T, HEADS, DH), axis=-1) / jnp.sqrt(float(DH))
    row = pl.program_id(0) * T + lax.broadcasted_iota(jnp.int32, (T, 1), 0)
    ex = jnp.where(row < E, jnp.exp(s), 0.0)
    ve = kvg[:, H:2 * H] + jnp.dot(he, wv)
    wv_o[...] = (ve.reshape(T, HEADS, DH) * ex[:, :, None]).reshape(T, H)
    e_o[...] = jnp.concatenate(
        [ex, jnp.zeros((T, W_NARROW - HEADS), jnp.float32)], axis=-1)


def _k_attn_edge_sv(qg_ref, kvtab_ref, si_ref, he_ref, wk_ref, wv_ref,
                    e_o, wv_o):
    oh = _onehot(si_ref[...], kvtab_ref.shape[0])
    kvg = jnp.dot(oh, kvtab_ref[...])
    _attn_edge_core(qg_ref[...], kvg, he_ref[...],
                    wk_ref[...], wv_ref[...], e_o, wv_o)


def _k_attn_edge_sq(qtab_ref, si_ref, kvg_ref, he_ref, wk_ref, wv_ref,
                    e_o, wv_o):
    oh = _onehot(si_ref[...], qtab_ref.shape[0])
    qg = jnp.dot(oh, qtab_ref[...])
    _attn_edge_core(qg, kvg_ref[...], he_ref[...],
                    wk_ref[...], wv_ref[...], e_o, wv_o)


def _k_comb(aggp_ref, zp_ref, hprev_ref, wo_ref, g_ref, b_ref, o_ref):
    T = hprev_ref.shape[0]
    aggp = aggp_ref[...]
    zp = zp_ref[...]
    z = (zp[0] + zp[1])[:, :HEADS]
    s = aggp[0] + aggp[1]
    agg = (s.reshape(T, HEADS, DH) / (z[:, :, None] + 1e-9)).reshape(T, H)
    h = hprev_ref[...] + jnp.dot(agg, wo_ref[...])
    o_ref[...] = _ln_k(h, g_ref[...], b_ref[...])


def _k_delta(hvtab_ref, si_ref, hnd_ref, he_ref, ed_ref, bias_ref,
             w1a, w1b, w1c, w1d, w1e, b1, w2, b2, o_ref):
    oh = _onehot(si_ref[...], hvtab_ref.shape[0])
    tv = jnp.dot(hvtab_ref[...], w1a[...])
    x = (jnp.dot(oh, tv) + jnp.dot(hnd_ref[...], w1b[...])
         + jnp.dot(he_ref[...], w1c[...]) + jnp.dot(ed_ref[...], w1d[...])
         + bias_ref[...] * w1e[...] + b1[...])
    o_ref[...] = bias_ref[...] + jnp.dot(_silu(x), w2[...]) + b2[...]


def _k_glob(hg_ref, pooledp_ref, cntp_ref, temb_ref,
            w1a, w1b, w1c, b1, w2, b2, o_ref):
    pooledp = pooledp_ref[...]
    cntp = cntp_ref[...]
    cnt = (cntp[0] + cntp[1])[:, :1] + 1e-6
    pooled = (pooledp[0] + pooledp[1]) / cnt
    hg = hg_ref[...]
    x = (jnp.dot(hg, w1a[...]) + jnp.dot(pooled, w1b[...])
         + jnp.dot(temb_ref[...], w1c[...]) + b1[...])
    o_ref[...] = hg + jnp.dot(_silu(x), w2[...]) + b2[...]


def _k_final(hnd_ref, hvtab_ref, si_ref, he_ref, edp_ref, gt_ref, gi_ref,
             bias_ref, lng, lnb, w1, b1, w2, b2, w3, b3, o_ref):
    he = _ln_k(he_ref[...], lng[...], lnb[...])
    hnd = hnd_ref[...]
    hvs = jnp.dot(_onehot(si_ref[...], hvtab_ref.shape[0]), hvtab_ref[...])
    w = w1[...]
    g2 = jnp.dot(gt_ref[...], w[512:768])
    x = (jnp.dot(hnd, w[0:128]) + jnp.dot(hvs, w[128:256])
         + jnp.dot(he, w[256:384]) + jnp.dot(edp_ref[...], w[384:512])
         + jnp.dot(_onehot(gi_ref[...], B), g2)
         + jnp.dot(hnd * hvs, w[768:896]) + b1[...])
    x = _silu(x)
    x = _silu(jnp.dot(x, w2[...]) + b2[...])
    o_ref[...] = jnp.dot(x, w3[...]) + b3[...] + bias_ref[...]


def _r1(x):
    return jnp.asarray(x, jnp.float32).reshape(1, -1)


def _mlp2_args(p):
    return (p["w1"], _r1(p["b1"]), p["w2"], _r1(p["b2"]))


def kernel(node_feat, veh_feat, edge_attr, xt01, t, graph_feat, edge_dyn,
           params, src, dst, edge_graph, node_batch):
    p = params
    f32 = jnp.float32
    src = src.astype(jnp.int32)
    dst = dst.astype(jnp.int32)
    edge_graph = edge_graph.astype(jnp.int32)
    node_batch = node_batch.astype(jnp.int32)

    def padE(a):
        pw = ((0, E_PAD - E),) + ((0, 0),) * (a.ndim - 1)
        return jnp.pad(a, pw)

    srcp = padE(src).reshape(-1, 128)
    dstp = padE(dst).reshape(-1, 128)
    dstp64 = padE(dst).reshape(-1, 64)
    srci = padE(src).reshape(E_PAD, 1)
    egi = padE(edge_graph).reshape(E_PAD, 1)
    eap = padE(jnp.asarray(edge_attr, f32))
    xtp = padE(jnp.asarray(xt01, f32)).reshape(E_PAD, 1)
    edp = padE(jnp.asarray(edge_dyn, f32))
    nbp = jnp.pad(node_batch, (0, N_PAD - N))
    ones_rows = jnp.zeros((N_PAD, W_NARROW), f32).at[:N, :].set(1.0)

    tcol = jnp.asarray(t, f32).reshape(B, 1)
    half = TIME_DIM // 2
    freqs = jnp.exp(-jnp.log(10000.0)
                    * jnp.arange(half, dtype=f32) / float(half)).reshape(1, half)

    temb, h_g, h_v = pl.pallas_call(
        _k_smalls,
        in_specs=[_full((B, 1)), _full((1, half)), _full((B, 8)),
                  _full((V, 8)),
                  _full((TIME_DIM, H)), _full((1, H)), _full((H, H)), _full((1, H)),
                  _full((8, H)), _full((1, H)), _full((H, H)), _full((1, H)),
                  _full((8, H)), _full((1, H)), _full((H, H)), _full((1, H))],
        out_specs=[_full((B, H)), _full((B, H)), _full((V, H))],
        out_shape=[jax.ShapeDtypeStruct((B, H), f32),
                   jax.ShapeDtypeStruct((B, H), f32),
                   jax.ShapeDtypeStruct((V, H), f32)],
    )(tcol, freqs, jnp.asarray(graph_feat, f32), jnp.asarray(veh_feat, f32),
      *_mlp2_args(p["time_proj"]), *_mlp2_args(p["global_proj"]),
      *_mlp2_args(p["veh_proj"]))

    h_n = pl.pallas_call(
        _k_mlp2,
        grid=(N // NT,),
        in_specs=[_rows(NT, (N, 8)), _full((8, H)), _full((1, H)),
                  _full((H, H)), _full((1, H))],
        out_specs=_rows(NT, (N, H)),
        out_shape=jax.ShapeDtypeStruct((N, H), f32),
    )(jnp.asarray(node_feat, f32), *_mlp2_args(p["node_proj"]))

    ep = p["edge_proj"]
    h_e = pl.pallas_call(
        _k_eproj,
        grid=(E_PAD // ET,),
        in_specs=[_rows(ET, (E_PAD, 4)), _rows(ET, (E_PAD, 1)),
                  _full((4, H)), _full((1, H)), _full((1, H)),
                  _full((H, H)), _full((1, H))],
        out_specs=_rows(ET, (E_PAD, H)),
        out_shape=jax.ShapeDtypeStruct((E_PAD, H), f32),
    )(eap, xtp, ep["w1"][:4], _r1(ep["w1"][4]), _r1(ep["b1"]),
      ep["w2"], _r1(ep["b2"]))

    bp = p["edge_bias_mlp"]
    e_dyn, bias = pl.pallas_call(
        _k_edyn,
        grid=(E_PAD // ET,),
        in_specs=[_rows(ET, (E_PAD, 7)),
                  _full((7, H)), _full((1, H)), _full((H, H)), _full((1, H)),
                  _full((7, H // 2)), _full((1, H // 2)),
                  _full((H // 2, 1)), _full((1, 1))],
        out_specs=[_rows(ET, (E_PAD, H)), _rows(ET, (E_PAD, 1))],
        out_shape=[jax.ShapeDtypeStruct((E_PAD, H), f32),
                   jax.ShapeDtypeStruct((E_PAD, 1), f32)],
    )(edp, *_mlp2_args(p["edge_dyn_proj"]), *_mlp2_args(bp))

    cntp = _scatter_add_small(ones_rows, nbp, B, c=64)

    def _mm_t(x, w):
        r0, wb = x.shape[0], w.shape[1]
        rt0 = min(r0, NT)
        return pl.pallas_call(
            _k_mm, grid=(r0 // rt0,),
            in_specs=[_rows(rt0, (r0, H)), _full((H, wb))],
            out_specs=_rows(rt0, (r0, wb)),
            out_shape=jax.ShapeDtypeStruct((r0, wb), f32),
        )(x, w)

    def attn(h_dst_tab, small_tab, wide_arr, wide_spec, q_idx, r,
             ap, lnp, si, sv):
        rt = min(r, NT)
        outs = [_rows(ET, (E_PAD, W_NARROW)), _rows(ET, (E_PAD, H))]
        oshape = [jax.ShapeDtypeStruct((E_PAD, W_NARROW), f32),
                  jax.ShapeDtypeStruct((E_PAD, H), f32)]
        if sv:
            e16, wvrows = pl.pallas_call(
                _k_attn_edge_sv, grid=(E_PAD // ET,),
                in_specs=[wide_spec, _full((V, 2 * H)),
                          _rows(ET, (E_PAD, 1)), _rows(ET, (E_PAD, H)),
                          _full((H, H)), _full((H, H))],
                out_specs=outs, out_shape=oshape,
            )(wide_arr, small_tab, si, h_e, ap["wk"], ap["wv"])
        else:
            e16, wvrows = pl.pallas_call(
                _k_attn_edge_sq, grid=(E_PAD // ET,),
                in_specs=[_full((V, H)), _rows(ET, (E_PAD, 1)),
                          wide_spec, _rows(ET, (E_PAD, H)),
                          _full((H, H)), _full((H, H))],
                out_specs=outs, out_shape=oshape,
            )(small_tab, si, wide_arr, h_e, ap["wk"], ap["wv"])

        zp = _scatter_add(e16, q_idx, r)
        aggp = _scatter_add(wvrows, q_idx, r)

        h_new = pl.pallas_call(
            _k_comb, grid=(r // rt,),
            in_specs=[pl.BlockSpec((2, rt, H), lambda i: (0, i, 0)),
                      pl.BlockSpec((2, rt, W_NARROW), lambda i: (0, i, 0)),
                      _rows(rt, (r, H)), _full((H, H)),
                      _full((1, H)), _full((1, H))],
            out_specs=_rows(rt, (r, H)),
            out_shape=jax.ShapeDtypeStruct((r, H), f32),
        )(aggp, zp, h_dst_tab, ap["wo"], _r1(lnp["g"]), _r1(lnp["b"]))
        return h_new

    layers = p["layers"]
    nl = len(layers)
    eyeH = jnp.eye(H, dtype=f32)
    qg_a = _gather(_mm_t(h_n, layers[0]["v2n"]["wq"]), dstp)
    qg_s = _rows(ET, (E_PAD, H))
    for li, lp in enumerate(layers):
        ap1, ap2 = lp["v2n"], lp["n2v"]
        kvtab_v = _mm_t(h_v, jnp.concatenate([ap1["wk"], ap1["wv"]], axis=1))
        h_n = attn(h_n, kvtab_v, qg_a, qg_s, dstp, N,
                   ap1, lp["ln_n"], srci, True)

        cols = [ap2["wk"], ap2["wv"], eyeH]
        if li + 1 < nl:
            cols.append(layers[li + 1]["v2n"]["wq"])
        bigg = _gather(_mm_t(h_n, jnp.concatenate(cols, axis=1)), dstp64)
        hnd_s = pl.BlockSpec((ET, H), lambda i: (i, 2))

        qtab_v = _mm_t(h_v, ap2["wq"])
        h_v = attn(h_v, qtab_v, bigg,
                   pl.BlockSpec((ET, 2 * H), lambda i: (i, 0)),
                   srcp, V, ap2, lp["ln_v"], srci, False)
        if li + 1 < nl:
            qg_a = bigg
            qg_s = pl.BlockSpec((ET, H), lambda i: (i, 3))

        dp = lp["edge_delta"]
        w1 = dp["w1"]
        bias = pl.pallas_call(
            _k_delta, grid=(E_PAD // ET,),
            in_specs=[_full((V, H)), _rows(ET, (E_PAD, 1)),
                      hnd_s,
                      _rows(ET, (E_PAD, H)), _rows(ET, (E_PAD, 7)),
                      _rows(ET, (E_PAD, 1)),
                      _full((H, H // 2)), _full((H, H // 2)),
                      _full((H, H // 2)), _full((7, H // 2)),
                      _full((1, H // 2)), _full((1, H // 2)),
                      _full((H // 2, 1)), _full((1, 1))],
            out_specs=_rows(ET, (E_PAD, 1)),
            out_shape=jax.ShapeDtypeStruct((E_PAD, 1), f32),
        )(h_v, srci, bigg, h_e, edp, bias,
          w1[0:H], w1[H:2 * H], w1[2 * H:3 * H], w1[3 * H:3 * H + 7],
          _r1(w1[3 * H + 7]), _r1(dp["b1"]), dp["w2"], _r1(dp["b2"]))

        hn_pad = jnp.pad(h_n, ((0, N_PAD - N), (0, 0)))
        pooledp = _scatter_add_small(hn_pad, nbp, B, c=64)
        gp = lp["glob"]
        gw1 = gp["w1"]
        h_g = pl.pallas_call(
            _k_glob,
            grid=(1,),
            in_specs=[_full((B, H)), _full((2, B, H)),
                      _full((2, B, W_NARROW)), _full((B, H)),
                      _full((H, H)), _full((H, H)), _full((H, H)),
                      _full((1, H)), _full((H, H)), _full((1, H))],
            out_specs=_full((B, H)),
            out_shape=jax.ShapeDtypeStruct((B, H), f32),
        )(h_g, pooledp, cntp, temb,
          gw1[0:H], gw1[H:2 * H], gw1[2 * H:3 * H], _r1(gp["b1"]),
          gp["w2"], _r1(gp["b2"]))

    gt_tab = jnp.concatenate([h_g, temb], axis=1)
    ph = p["edge_head"]
    oln = p["edge_out_ln"]
    logits = pl.pallas_call(
        _k_final, grid=(E_PAD // ET,),
        in_specs=[hnd_s, _full((V, H)),
                  _rows(ET, (E_PAD, 1)),
                  _rows(ET, (E_PAD, H)), _rows(ET, (E_PAD, H)),
                  _full((B, 2 * H)), _rows(ET, (E_PAD, 1)),
                  _rows(ET, (E_PAD, 1)),
                  _full((1, H)), _full((1, H)),
                  _full((7 * H, H)), _full((1, H)),
                  _full((H, H // 2)), _full((1, H // 2)),
                  _full((H // 2, 1)), _full((1, 1))],
        out_specs=_rows(ET, (E_PAD, 1)),
        out_shape=jax.ShapeDtypeStruct((E_PAD, 1), f32),
    )(bigg, h_v, srci, h_e, e_dyn, gt_tab, egi, bias,
      _r1(oln["g"]), _r1(oln["b"]), ph["w1"], _r1(ph["b1"]),
      ph["w2"], _r1(ph["b2"]), ph["w3"], _r1(ph["b3"]))

    return logits[:E, 0]
